# baseline (device time: 6693 ns/iter reference)
import jax
import jax.numpy as jnp
from jax import lax
from jax.experimental import pallas as pl
from jax.experimental.pallas import tpu as pltpu

N_DEV = 8


def kernel(x):
    m, n = x.shape

    def body(x_ref, out_ref):
        my_pos = lax.axis_index("i")
        barrier_sem = pltpu.get_barrier_semaphore()
        for d in range(1, N_DEV):
            peer = lax.rem(my_pos + d, N_DEV)
            pl.semaphore_signal(
                barrier_sem, inc=1,
                device_id=(peer,), device_id_type=pl.DeviceIdType.MESH,
            )
        xv = x_ref[:, :]
        mx = jnp.max(xv, axis=0, keepdims=True)
        ids = lax.broadcasted_iota(jnp.int32, (m, n), 0)
        loc = jnp.min(jnp.where(xv == mx, ids, m), axis=0, keepdims=True)
        gidx = (loc + my_pos * m).astype(jnp.float32)
        pl.semaphore_wait(barrier_sem, N_DEV - 1)
        out_ref[0:1, :] = mx
        out_ref[1:2, :] = gidx

    return pl.pallas_call(
        body,
        out_shape=jax.ShapeDtypeStruct((2, n), jnp.float32),
        in_specs=[pl.BlockSpec(memory_space=pltpu.VMEM)],
        out_specs=pl.BlockSpec(memory_space=pltpu.VMEM),
        compiler_params=pltpu.CompilerParams(collective_id=0),
    )(x)
